# baseline (speedup 1.0000x reference)
import jax
import jax.numpy as jnp
from jax import lax
from jax.experimental import pallas as pl
from jax.experimental.pallas import tpu as pltpu, tpu_sc as plsc

EMB = 64
NC, NS = 2, 16
NW = NC * NS
BW = 128
NBUF = 4


def _build(n_batch, seq):
    n_bt = n_batch // BW
    bt_per_w = n_bt // NW
    n_groups = seq
    mesh = plsc.VectorSubcoreMesh(core_axis_name="c", subcore_axis_name="s")

    def body(x_hbm, table_hbm, out_hbm, xv, *bufs):
        gbuf = bufs[:NBUF]
        tbuf = bufs[NBUF:2 * NBUF]
        sem_g = bufs[2 * NBUF:3 * NBUF]
        sem_o = bufs[3 * NBUF:]
        wid = lax.axis_index("s") * NC + lax.axis_index("c")
        bt0 = wid * bt_per_w

        pltpu.sync_copy(x_hbm.at[:, pl.ds(bt0, bt_per_w)], xv)

        def gather_copy(l, j, b):
            return pltpu.make_async_copy(
                table_hbm.at[xv.at[l, j]], gbuf[b], sem_g[b]
            )

        def store_copy(l, j, b):
            return pltpu.make_async_copy(
                tbuf[b], out_hbm.at[l, :, bt0 + j], sem_o[b]
            )

        def transpose(b):
            def krow(k, carry):
                rows = lax.iota(jnp.int32, 16) + k * 16
                for e in range(EMB):
                    cols = jnp.full((16,), e, jnp.int32)
                    vec = plsc.load_gather(gbuf[b], [rows, cols])
                    tbuf[b][e // 8, e % 8, pl.ds(k * 16, 16)] = vec
                return carry

            lax.fori_loop(0, BW // 16, krow, 0)

        for b in range(NBUF):
            gather_copy(0, b, b).start()

        def group(l, carry):
            for b in range(NBUF):
                gather_copy(l, b, b).wait()

                @pl.when(l > 0)
                def _():
                    store_copy(l - 1, b, b).wait()

                transpose(b)

                @pl.when(l < n_groups - 1)
                def _():
                    gather_copy(l + 1, b, b).start()

                store_copy(l, b, b).start()
            return carry

        lax.fori_loop(0, n_groups, group, 0)

        for b in range(NBUF):
            store_copy(n_groups - 1, b, b).wait()

    return pl.kernel(
        body,
        out_type=jax.ShapeDtypeStruct(
            (seq, EMB // 8, n_bt, 8, BW), jnp.float32
        ),
        mesh=mesh,
        scratch_types=[pltpu.VMEM((seq, bt_per_w, BW), jnp.int32)]
        + [pltpu.VMEM((BW, EMB), jnp.float32)] * NBUF
        + [pltpu.VMEM((EMB // 8, 8, BW), jnp.float32)] * NBUF
        + [pltpu.SemaphoreType.DMA] * (2 * NBUF),
        compiler_params=pltpu.CompilerParams(
            use_tc_tiling_on_sc=False, needs_layout_passes=False
        ),
    )


def kernel(x, table):
    B, L = x.shape
    x3 = jnp.transpose(x).reshape(L, B // BW, BW).astype(jnp.int32)
    out5 = _build(B, L)(x3, table)
    return out5.transpose(2, 4, 0, 1, 3).reshape(B, L, EMB)


# speedup vs baseline: 1.4617x; 1.4617x over previous
import jax
import jax.numpy as jnp
from jax import lax
from jax.experimental import pallas as pl
from jax.experimental.pallas import tpu as pltpu, tpu_sc as plsc

EMB = 64
NC, NS = 2, 16
NW = NC * NS
BW = 128
NBUF = 4


def _build(n_batch, seq):
    n_bt = n_batch // BW
    bt_per_w = n_bt // NW
    n_groups = seq
    mesh = plsc.VectorSubcoreMesh(core_axis_name="c", subcore_axis_name="s")

    def body(x_hbm, table_hbm, out_hbm, xv, *bufs):
        gbuf = bufs[:NBUF]
        tbuf = bufs[NBUF:2 * NBUF]
        sem_g = bufs[2 * NBUF:3 * NBUF]
        sem_o = bufs[3 * NBUF:]
        wid = lax.axis_index("s") * NC + lax.axis_index("c")
        bt0 = wid * bt_per_w

        pltpu.sync_copy(x_hbm.at[:, pl.ds(bt0, bt_per_w)], xv)

        def gather_copy(l, j, b):
            return pltpu.make_async_copy(
                table_hbm.at[xv.at[l, j]], gbuf[b], sem_g[b]
            )

        def store_copy(l, j, b):
            return pltpu.make_async_copy(
                tbuf[b], out_hbm.at[l, :, bt0 + j], sem_o[b]
            )

        rows = [lax.iota(jnp.int32, 16) + k * 16 for k in range(BW // 16)]

        def transpose(b):
            @plsc.parallel_loop(0, EMB, unroll=4)
            def _(e):
                cols = jnp.full((16,), e, jnp.int32)
                et = e // 8
                ei = e % 8
                for k in range(BW // 16):
                    vec = plsc.load_gather(gbuf[b], [rows[k], cols])
                    tbuf[b][et, ei, pl.ds(k * 16, 16)] = vec

        for b in range(NBUF):
            gather_copy(0, b, b).start()

        def group(l, carry):
            for b in range(NBUF):
                gather_copy(l, b, b).wait()

                @pl.when(l > 0)
                def _():
                    store_copy(l - 1, b, b).wait()

                transpose(b)

                @pl.when(l < n_groups - 1)
                def _():
                    gather_copy(l + 1, b, b).start()

                store_copy(l, b, b).start()
            return carry

        lax.fori_loop(0, n_groups, group, 0)

        for b in range(NBUF):
            store_copy(n_groups - 1, b, b).wait()

    return pl.kernel(
        body,
        out_type=jax.ShapeDtypeStruct(
            (seq, EMB // 8, n_bt, 8, BW), jnp.float32
        ),
        mesh=mesh,
        scratch_types=[pltpu.VMEM((seq, bt_per_w, BW), jnp.int32)]
        + [pltpu.VMEM((BW, EMB), jnp.float32)] * NBUF
        + [pltpu.VMEM((EMB // 8, 8, BW), jnp.float32)] * NBUF
        + [pltpu.SemaphoreType.DMA] * (2 * NBUF),
        compiler_params=pltpu.CompilerParams(
            use_tc_tiling_on_sc=False, needs_layout_passes=False
        ),
    )


def kernel(x, table):
    B, L = x.shape
    x3 = jnp.transpose(x).reshape(L, B // BW, BW).astype(jnp.int32)
    out5 = _build(B, L)(x3, table)
    return out5.transpose(2, 4, 0, 1, 3).reshape(B, L, EMB)
